# baseline (speedup 1.0000x reference)
import jax
import jax.numpy as jnp
from jax.experimental import pallas as pl

BB = 1024


def _make_tc_body(V, KP, BB):
    def body(ids_ref, mask_ref, tab_ref, out_ref):
        ids = jnp.broadcast_to(ids_ref[0], (KP, BB))
        m = jnp.broadcast_to(mask_ref[0] == 0, (KP, BB))
        row = jax.lax.broadcasted_iota(jnp.int32, (KP, BB), 0)
        hot = ((row == ids) | (row == V)) & m
        a_t = jnp.where(hot, 1.0, 0.0).astype(jnp.float32)
        out_ref[0, :, :] = jax.lax.dot_general(
            a_t, tab_ref[:, :], (((0,), (0,)), ((), ())),
            preferred_element_type=jnp.float32)
    return body


def kernel(phoneme_ids, padding_mask, table, pos_bias):
    B, T = phoneme_ids.shape
    V, D = table.shape
    N = B * T
    NB = N // BB
    KP = ((V + 1 + 7) // 8) * 8

    ids = phoneme_ids.reshape(NB, 1, BB).astype(jnp.int32)
    mask = padding_mask.reshape(NB, 1, BB).astype(jnp.int32)
    aug = jnp.concatenate(
        [table.astype(jnp.float32),
         pos_bias.reshape(1, D).astype(jnp.float32),
         jnp.zeros((KP - V - 1, D), jnp.float32)], axis=0)

    out = pl.pallas_call(
        _make_tc_body(V, KP, BB),
        out_shape=jax.ShapeDtypeStruct((NB, BB, D), jnp.float32),
        grid=(NB,),
        in_specs=[
            pl.BlockSpec((1, 1, BB), lambda i: (i, 0, 0)),
            pl.BlockSpec((1, 1, BB), lambda i: (i, 0, 0)),
            pl.BlockSpec((KP, D), lambda i: (0, 0)),
        ],
        out_specs=pl.BlockSpec((1, BB, D), lambda i: (i, 0, 0)),
    )(ids, mask, aug)
    return out.reshape(B, T, D)
